# baseline (speedup 1.0000x reference)
import functools

import jax
import jax.numpy as jnp
from jax import lax
from jax.experimental import pallas as pl
from jax.experimental.pallas import tpu as pltpu
from jax.experimental.pallas import tpu_sc as plsc

_N_CODES = 512
_CODE_DIM = 64
_H_TILE = 32

_NC = 2
_NS = 16
_NW = _NC * _NS
_CHUNK = 128


def _vq_tc_body(z_ref, cb_ref, kiota_ref, idx_ref, acc_ref):
    x = z_ref[0].reshape(_CODE_DIM, _H_TILE * 64)
    cb = cb_ref[...]
    cnorm = jnp.sum(cb * cb, axis=1)
    znorm = jnp.sum(x * x, axis=0)
    s2 = lax.dot_general(cb * (-2.0), x, (((1,), (0,)), ((), ())),
                         preferred_element_type=jnp.float32)
    dist = (znorm[None, :] + s2) + cnorm[:, None]
    m = jnp.min(dist, axis=0)
    idx_f = jnp.min(jnp.where(dist == m[None, :], kiota_ref[...],
                              jnp.float32(_N_CODES)), axis=0)
    idx = idx_f.astype(jnp.int32)
    idx_ref[0] = idx.reshape(idx_ref.shape[1:])

    @pl.when((pl.program_id(0) == 0) & (pl.program_id(1) == 0))
    def _():
        acc_ref[0, 0] = 0.0

    acc_ref[0, 0] += jnp.sum(m)


def _sc_gather_body(cb_hbm, idx_hbm, out_hbm, idx_v, rows_v, cb_sh, sem, *,
                    n_streams):
    w = lax.axis_index("s") * _NC + lax.axis_index("c")
    ns2 = n_streams // 2
    rows_per_half = ns2 * _CHUNK

    @pl.when(lax.axis_index("s") == 0)
    def _():
        pltpu.sync_copy(cb_hbm, cb_sh)

    pltpu.sync_copy(idx_hbm.at[w], idx_v)
    plsc.subcore_barrier()
    for half in range(2):
        descs = []
        for j in range(ns2):
            descs.append(pltpu.async_copy(
                cb_sh.at[idx_v.at[half * ns2 + j]],
                rows_v.at[pl.ds(j * _CHUNK, _CHUNK)], sem))
        for d in descs:
            d.wait()
        pltpu.sync_copy(
            rows_v, out_hbm.at[w, pl.ds(half * rows_per_half, rows_per_half)])


def _run_part(zh, codebook):
    Bh, C, H, W = zh.shape
    nh = H // _H_TILE
    n_tok = Bh * H * W
    idx3, acc = pl.pallas_call(
        _vq_tc_body,
        grid=(Bh, nh),
        in_specs=[
            pl.BlockSpec((1, C, _H_TILE, W), lambda b, h: (b, 0, h, 0)),
            pl.BlockSpec((_N_CODES, _CODE_DIM), lambda b, h: (0, 0)),
            pl.BlockSpec((_N_CODES, _H_TILE * W), lambda b, h: (0, 0)),
        ],
        out_specs=[
            pl.BlockSpec((1, _H_TILE * W // _CHUNK, _CHUNK),
                         lambda b, h, nh=nh: (b * nh + h, 0, 0)),
            pl.BlockSpec(memory_space=pltpu.SMEM),
        ],
        out_shape=[
            jax.ShapeDtypeStruct((Bh * nh, _H_TILE * W // _CHUNK, _CHUNK),
                                 jnp.int32),
            jax.ShapeDtypeStruct((1, 1), jnp.float32),
        ],
    )(zh, codebook,
      lax.broadcasted_iota(jnp.float32, (_N_CODES, _H_TILE * W), 0))

    per_w = n_tok // _NW
    sc_gather = pl.kernel(
        functools.partial(_sc_gather_body, n_streams=per_w // _CHUNK),
        out_type=jax.ShapeDtypeStruct((_NW, per_w, _CODE_DIM), jnp.float32),
        mesh=plsc.VectorSubcoreMesh(core_axis_name="c", subcore_axis_name="s"),
        scratch_types=[
            pltpu.VMEM((per_w // _CHUNK, _CHUNK), jnp.int32),
            pltpu.VMEM((per_w // 2, _CODE_DIM), jnp.float32),
            pltpu.VMEM_SHARED((_N_CODES, _CODE_DIM), jnp.float32),
            pltpu.SemaphoreType.DMA,
        ],
        compiler_params=pltpu.CompilerParams(use_tc_tiling_on_sc=False),
    )
    zq_rows = sc_gather(codebook, idx3.reshape(_NW, per_w // _CHUNK, _CHUNK))
    z_q_st = zq_rows.reshape(Bh, H, W, C).transpose(0, 3, 1, 2)
    return z_q_st, acc, idx3.reshape(Bh, H * W)


@jax.jit
def kernel(z, codebook):
    B, C, H, W = z.shape
    z_q_st, acc, indices = _run_part(z, codebook)
    vq_loss = acc[0, 0] * jnp.float32(1.25 / (B * C * H * W))
    return z_q_st, vq_loss, indices


# speedup vs baseline: 1.0292x; 1.0292x over previous
import functools

import jax
import jax.numpy as jnp
from jax import lax
from jax.experimental import pallas as pl
from jax.experimental.pallas import tpu as pltpu
from jax.experimental.pallas import tpu_sc as plsc

_N_CODES = 512
_CODE_DIM = 64
_H_TILE = 32

_NC = 2
_NS = 16
_NW = _NC * _NS
_CHUNK = 128


def _vq_tc_body(z_ref, cb_ref, idx_ref, acc_ref):
    x = z_ref[0].reshape(_CODE_DIM, _H_TILE * 64)
    cb = cb_ref[...]
    cnorm = jnp.sum(cb * cb, axis=1)
    znorm = jnp.sum(x * x, axis=0)
    s2 = lax.dot_general(cb * (-2.0), x, (((1,), (0,)), ((), ())),
                         preferred_element_type=jnp.float32)
    dist = (znorm[None, :] + s2) + cnorm[:, None]
    m = jnp.min(dist, axis=0)
    kiota = lax.broadcasted_iota(jnp.int32, dist.shape, 0)
    idx = jnp.min(jnp.where(dist == m[None, :], kiota, _N_CODES), axis=0)
    idx_ref[0] = idx.reshape(idx_ref.shape[1:])

    @pl.when((pl.program_id(0) == 0) & (pl.program_id(1) == 0))
    def _():
        acc_ref[0, 0] = 0.0

    acc_ref[0, 0] += jnp.sum(m)


def _sc_gather_body(cb_hbm, idx_hbm, out_hbm, idx_v, rows_a, rows_b, cb_sh,
                    sga, sgb, soa, sob, *, n_streams):
    w = lax.axis_index("s") * _NC + lax.axis_index("c")
    qc = n_streams // 4
    qrows = qc * _CHUNK

    @pl.when(lax.axis_index("s") == 0)
    def _():
        pltpu.sync_copy(cb_hbm, cb_sh)

    pltpu.sync_copy(idx_hbm.at[w], idx_v)
    plsc.subcore_barrier()

    bufs = [rows_a, rows_b]
    gsems = [sga, sgb]
    osems = [soa, sob]

    def fire(q):
        return [pltpu.async_copy(
            cb_sh.at[idx_v.at[q * qc + j]],
            bufs[q % 2].at[pl.ds(j * _CHUNK, _CHUNK)], gsems[q % 2])
            for j in range(qc)]

    def out(q):
        return pltpu.async_copy(
            bufs[q % 2], out_hbm.at[w, pl.ds(q * qrows, qrows)], osems[q % 2])

    g0 = fire(0)
    g1 = fire(1)
    for d in g0:
        d.wait()
    o0 = out(0)
    for d in g1:
        d.wait()
    o0.wait()
    g2 = fire(2)
    o1 = out(1)
    for d in g2:
        d.wait()
    o1.wait()
    g3 = fire(3)
    o2 = out(2)
    for d in g3:
        d.wait()
    o3 = out(3)
    o2.wait()
    o3.wait()


def _run_part(zh, codebook):
    Bh, C, H, W = zh.shape
    nh = H // _H_TILE
    n_tok = Bh * H * W
    idx3, acc = pl.pallas_call(
        _vq_tc_body,
        grid=(Bh, nh),
        in_specs=[
            pl.BlockSpec((1, C, _H_TILE, W), lambda b, h: (b, 0, h, 0)),
            pl.BlockSpec((_N_CODES, _CODE_DIM), lambda b, h: (0, 0)),
        ],
        out_specs=[
            pl.BlockSpec((1, _H_TILE * W // _CHUNK, _CHUNK),
                         lambda b, h, nh=nh: (b * nh + h, 0, 0)),
            pl.BlockSpec(memory_space=pltpu.SMEM),
        ],
        out_shape=[
            jax.ShapeDtypeStruct((Bh * nh, _H_TILE * W // _CHUNK, _CHUNK),
                                 jnp.int32),
            jax.ShapeDtypeStruct((1, 1), jnp.float32),
        ],
    )(zh, codebook)

    per_w = n_tok // _NW
    sc_gather = pl.kernel(
        functools.partial(_sc_gather_body, n_streams=per_w // _CHUNK),
        out_type=jax.ShapeDtypeStruct((_NW, per_w, _CODE_DIM), jnp.float32),
        mesh=plsc.VectorSubcoreMesh(core_axis_name="c", subcore_axis_name="s"),
        scratch_types=[
            pltpu.VMEM((per_w // _CHUNK, _CHUNK), jnp.int32),
            pltpu.VMEM((per_w // 4, _CODE_DIM), jnp.float32),
            pltpu.VMEM((per_w // 4, _CODE_DIM), jnp.float32),
            pltpu.VMEM_SHARED((_N_CODES, _CODE_DIM), jnp.float32),
            pltpu.SemaphoreType.DMA,
            pltpu.SemaphoreType.DMA,
            pltpu.SemaphoreType.DMA,
            pltpu.SemaphoreType.DMA,
        ],
        compiler_params=pltpu.CompilerParams(use_tc_tiling_on_sc=False),
    )
    zq_rows = sc_gather(codebook, idx3.reshape(_NW, per_w // _CHUNK, _CHUNK))
    z_q_st = zq_rows.reshape(Bh, H, W, C).transpose(0, 3, 1, 2)
    return z_q_st, acc, idx3.reshape(Bh, H * W)


@jax.jit
def kernel(z, codebook):
    B, C, H, W = z.shape
    z_q_st, acc, indices = _run_part(z, codebook)
    vq_loss = acc[0, 0] * jnp.float32(1.25 / (B * C * H * W))
    return z_q_st, vq_loss, indices


# speedup vs baseline: 1.0558x; 1.0259x over previous
import functools

import jax
import jax.numpy as jnp
from jax import lax
from jax.experimental import pallas as pl
from jax.experimental.pallas import tpu as pltpu
from jax.experimental.pallas import tpu_sc as plsc

_N_CODES = 512
_CODE_DIM = 64
_H_TILE = 32

_NC = 2
_NS = 16
_NW = _NC * _NS
_CHUNK = 128


def _vq_tc_body(z_ref, cb_ref, idx_ref, acc_ref):
    x = z_ref[0].reshape(_CODE_DIM, _H_TILE * 64)
    cb = cb_ref[...]
    cnorm = jnp.sum(cb * cb, axis=1)
    znorm = jnp.sum(x * x, axis=0)
    s2 = lax.dot_general(cb * (-2.0), x, (((1,), (0,)), ((), ())),
                         preferred_element_type=jnp.float32)
    dist = (znorm[None, :] + s2) + cnorm[:, None]
    m = jnp.min(dist, axis=0)
    kiota = lax.broadcasted_iota(jnp.int32, dist.shape, 0).astype(jnp.float32)
    idx_f = jnp.min(jnp.where(dist == m[None, :], kiota,
                              jnp.float32(_N_CODES)), axis=0)
    idx_ref[0] = idx_f.astype(jnp.int32).reshape(idx_ref.shape[1:])

    @pl.when((pl.program_id(0) == 0) & (pl.program_id(1) == 0))
    def _():
        acc_ref[0, 0] = 0.0

    acc_ref[0, 0] += jnp.sum(m)


def _sc_gather_body(cb_hbm, idx_hbm, out_hbm, idx_v, rows_a, rows_b, cb_sh,
                    sga, sgb, soa, sob, *, n_streams):
    w = lax.axis_index("s") * _NC + lax.axis_index("c")
    qc = n_streams // 4
    qrows = qc * _CHUNK

    @pl.when(lax.axis_index("s") == 0)
    def _():
        pltpu.sync_copy(cb_hbm, cb_sh)

    pltpu.sync_copy(idx_hbm.at[w], idx_v)
    plsc.subcore_barrier()

    bufs = [rows_a, rows_b]
    gsems = [sga, sgb]
    osems = [soa, sob]

    def fire(q):
        return [pltpu.async_copy(
            cb_sh.at[idx_v.at[q * qc + j]],
            bufs[q % 2].at[pl.ds(j * _CHUNK, _CHUNK)], gsems[q % 2])
            for j in range(qc)]

    def out(q):
        return pltpu.async_copy(
            bufs[q % 2], out_hbm.at[w, pl.ds(q * qrows, qrows)], osems[q % 2])

    g0 = fire(0)
    g1 = fire(1)
    for d in g0:
        d.wait()
    o0 = out(0)
    for d in g1:
        d.wait()
    o0.wait()
    g2 = fire(2)
    o1 = out(1)
    for d in g2:
        d.wait()
    o1.wait()
    g3 = fire(3)
    o2 = out(2)
    for d in g3:
        d.wait()
    o3 = out(3)
    o2.wait()
    o3.wait()


def _run_part(zh, codebook):
    Bh, C, H, W = zh.shape
    nh = H // _H_TILE
    n_tok = Bh * H * W
    idx3, acc = pl.pallas_call(
        _vq_tc_body,
        grid=(Bh, nh),
        in_specs=[
            pl.BlockSpec((1, C, _H_TILE, W), lambda b, h: (b, 0, h, 0)),
            pl.BlockSpec((_N_CODES, _CODE_DIM), lambda b, h: (0, 0)),
        ],
        out_specs=[
            pl.BlockSpec((1, _H_TILE * W // _CHUNK, _CHUNK),
                         lambda b, h, nh=nh: (b * nh + h, 0, 0)),
            pl.BlockSpec(memory_space=pltpu.SMEM),
        ],
        out_shape=[
            jax.ShapeDtypeStruct((Bh * nh, _H_TILE * W // _CHUNK, _CHUNK),
                                 jnp.int32),
            jax.ShapeDtypeStruct((1, 1), jnp.float32),
        ],
    )(zh, codebook)

    per_w = n_tok // _NW
    sc_gather = pl.kernel(
        functools.partial(_sc_gather_body, n_streams=per_w // _CHUNK),
        out_type=jax.ShapeDtypeStruct((_NW, per_w, _CODE_DIM), jnp.float32),
        mesh=plsc.VectorSubcoreMesh(core_axis_name="c", subcore_axis_name="s"),
        scratch_types=[
            pltpu.VMEM((per_w // _CHUNK, _CHUNK), jnp.int32),
            pltpu.VMEM((per_w // 4, _CODE_DIM), jnp.float32),
            pltpu.VMEM((per_w // 4, _CODE_DIM), jnp.float32),
            pltpu.VMEM_SHARED((_N_CODES, _CODE_DIM), jnp.float32),
            pltpu.SemaphoreType.DMA,
            pltpu.SemaphoreType.DMA,
            pltpu.SemaphoreType.DMA,
            pltpu.SemaphoreType.DMA,
        ],
        compiler_params=pltpu.CompilerParams(use_tc_tiling_on_sc=False),
    )
    zq_rows = sc_gather(codebook, idx3.reshape(_NW, per_w // _CHUNK, _CHUNK))
    z_q_st = zq_rows.reshape(Bh, H, W, C).transpose(0, 3, 1, 2)
    return z_q_st, acc, idx3.reshape(Bh, H * W)


@jax.jit
def kernel(z, codebook):
    B, C, H, W = z.shape
    z_q_st, acc, indices = _run_part(z, codebook)
    vq_loss = acc[0, 0] * jnp.float32(1.25 / (B * C * H * W))
    return z_q_st, vq_loss, indices
